# baseline (device time: 149946 ns/iter reference)
import os

import jax
import jax.numpy as jnp
from jax import lax
from jax.experimental import pallas as pl
from jax.experimental.pallas import tpu as pltpu

_EXP = os.environ.get("SCB_EXP", "")

M = 2048
CHUNK = 2048
STRIP = 128

AXES = ("x", "y", "z")
GROUPS = ((0, 768), (768, 640), (1408, 640))
ORDERS = ((0, 1, 2), (1, 2, 0), (2, 0, 1))


def kernel(dy, W):
    recv_off = []
    recv_rows = []
    for _, rows in GROUPS:
        sizes = [rows // 2, rows // 4, rows // 8]
        offs = []
        o = 0
        for s in sizes:
            offs.append(o)
            o += s
        recv_off.append(offs)
        recv_rows.append(o)

    def body(dy_ref, w_ref, out_ref, acc_ref, b_ref, a_ref, ld_ref,
             rv0, rv1, rv2, ld_sems, send_sems, recv_sems):
        r = lax.axis_index("y") * 2 + lax.axis_index("z")
        col0 = r * CHUNK
        m = [lax.axis_index(ax) for ax in AXES]

        def nbr(ax):
            t = list(m)
            t[ax] = 1 - t[ax]
            return tuple(t)

        if _EXP != "matmul_only":
            barrier_sem = pltpu.get_barrier_semaphore()
            for ax in range(3):
                pl.semaphore_signal(
                    barrier_sem, inc=1, device_id=nbr(ax),
                    device_id_type=pl.DeviceIdType.MESH,
                )
            pl.semaphore_wait(barrier_sem, 3)

        def load(src_ref, row0, slot):
            cp = pltpu.make_async_copy(
                src_ref.at[pl.ds(row0, STRIP), pl.ds(col0, CHUNK)],
                ld_ref.at[slot],
                ld_sems.at[slot],
            )
            cp.start()
            return cp

        n_strips = M // STRIP
        cps = [None, None]
        cps[0] = load(w_ref, 0, 0)
        for s in range(n_strips):
            if s + 1 < n_strips:
                cps[(s + 1) % 2] = load(w_ref, (s + 1) * STRIP, (s + 1) % 2)
            cps[s % 2].wait()
            b_ref[s * STRIP:(s + 1) * STRIP, :] = ld_ref[s % 2].astype(
                jnp.bfloat16
            )

        recv_bufs = (rv0, rv1, rv2)
        starts = [GROUPS[g][0] for g in range(3)]
        sizes = [GROUPS[g][1] for g in range(3)]
        rdmas = [None, None, None]

        cps[0] = load(dy_ref, 0, 0)
        s_global = 0
        for g in range(3):
            gs, gr = GROUPS[g]
            for s in range(gr // STRIP):
                row0 = gs + s * STRIP
                if row0 + STRIP < M:
                    cps[(s_global + 1) % 2] = load(
                        dy_ref, row0 + STRIP, (s_global + 1) % 2
                    )
                cps[s_global % 2].wait()
                a_ref[...] = ld_ref[s_global % 2].astype(jnp.bfloat16)
                acc_ref[row0:row0 + STRIP, :] = lax.dot_general(
                    a_ref[...],
                    b_ref[...],
                    dimension_numbers=(((1,), (1,)), ((), ())),
                    preferred_element_type=jnp.float32,
                ).astype(jnp.bfloat16)
                s_global += 1
            if _EXP == "matmul_only":
                continue
            ax = ORDERS[g][0]
            bit = m[ax]
            half = gr // 2
            rdma = pltpu.make_async_remote_copy(
                src_ref=acc_ref.at[pl.ds(gs + (1 - bit) * half, half)],
                dst_ref=recv_bufs[g].at[pl.ds(recv_off[g][0], half)],
                send_sem=send_sems.at[g * 6],
                recv_sem=recv_sems.at[g * 6],
                device_id=nbr(ax),
                device_id_type=pl.DeviceIdType.MESH,
            )
            rdma.start()
            rdmas[g] = rdma
            starts[g] = gs + bit * half
            sizes[g] = half

        if _EXP == "matmul_only":
            out_ref[...] = acc_ref[...].astype(jnp.float32)
            return

        for g in range(3):
            rdmas[g].wait()
            acc_ref[pl.ds(starts[g], sizes[g]), :] += recv_bufs[g][
                pl.ds(recv_off[g][0], sizes[g]), :
            ]

        for rnd in range(1, 3):
            for g in range(3):
                ax = ORDERS[g][rnd]
                bit = m[ax]
                half = sizes[g] // 2
                rdma = pltpu.make_async_remote_copy(
                    src_ref=acc_ref.at[
                        pl.ds(starts[g] + (1 - bit) * half, half)
                    ],
                    dst_ref=recv_bufs[g].at[pl.ds(recv_off[g][rnd], half)],
                    send_sem=send_sems.at[g * 6 + rnd],
                    recv_sem=recv_sems.at[g * 6 + rnd],
                    device_id=nbr(ax),
                    device_id_type=pl.DeviceIdType.MESH,
                )
                rdma.start()
                rdmas[g] = rdma
                starts[g] = starts[g] + bit * half
                sizes[g] = half
            for g in range(3):
                rdmas[g].wait()
                acc_ref[pl.ds(starts[g], sizes[g]), :] += recv_bufs[g][
                    pl.ds(recv_off[g][rnd], sizes[g]), :
                ]

        other_starts = [None, None, None]
        pend = [None, None, None]
        for j in range(3):
            for g in range(3):
                ax = ORDERS[g][2 - j]
                bit = m[ax]
                sz = sizes[g]
                old = starts[g]
                rdma = pltpu.make_async_remote_copy(
                    src_ref=acc_ref.at[pl.ds(old, sz)],
                    dst_ref=acc_ref.at[pl.ds(old, sz)],
                    send_sem=send_sems.at[g * 6 + 3 + j],
                    recv_sem=recv_sems.at[g * 6 + 3 + j],
                    device_id=nbr(ax),
                    device_id_type=pl.DeviceIdType.MESH,
                )
                rdma.start()
                rdmas[g] = rdma
                other_starts[g] = old + (1 - 2 * bit) * sz
                starts[g] = old - bit * sz
                if j == 0:
                    pend[g] = (old, sz)
            for g in range(3):
                sz = sizes[g]
                ps, prows = pend[g]
                out_ref[pl.ds(ps, prows), :] = acc_ref[
                    pl.ds(ps, prows), :
                ].astype(jnp.float32)
                rdmas[g].wait()
                pend[g] = (other_starts[g], sz)
                sizes[g] = 2 * sz
        for g in range(3):
            ps, prows = pend[g]
            out_ref[pl.ds(ps, prows), :] = acc_ref[
                pl.ds(ps, prows), :
            ].astype(jnp.float32)

    return pl.pallas_call(
        body,
        out_shape=jax.ShapeDtypeStruct((M, M), jnp.float32),
        in_specs=[
            pl.BlockSpec(memory_space=pl.MemorySpace.ANY),
            pl.BlockSpec(memory_space=pl.MemorySpace.ANY),
        ],
        out_specs=pl.BlockSpec(memory_space=pltpu.VMEM),
        scratch_shapes=[
            pltpu.VMEM((M, M), jnp.bfloat16),
            pltpu.VMEM((M, CHUNK), jnp.bfloat16),
            pltpu.VMEM((STRIP, CHUNK), jnp.bfloat16),
            pltpu.VMEM((2, STRIP, CHUNK), jnp.float32),
            pltpu.VMEM((recv_rows[0], M), jnp.bfloat16),
            pltpu.VMEM((recv_rows[1], M), jnp.bfloat16),
            pltpu.VMEM((recv_rows[2], M), jnp.bfloat16),
            pltpu.SemaphoreType.DMA((2,)),
            pltpu.SemaphoreType.DMA((18,)),
            pltpu.SemaphoreType.DMA((18,)),
        ],
        compiler_params=pltpu.CompilerParams(
            collective_id=None if _EXP == "matmul_only" else 0,
            vmem_limit_bytes=63 * 1024 * 1024,
        ),
    )(dy, W)
